# Initial kernel scaffold; baseline (speedup 1.0000x reference)
import jax
import jax.numpy as jnp
from jax.experimental import pallas as pl


def kernel(phases, S, G):
    raise NotImplementedError("write your pallas kernel here")


# speedup vs baseline: 1.3790x; 1.3790x over previous
import functools

import jax
import jax.numpy as jnp
from jax.experimental import pallas as pl
from jax.experimental.pallas import tpu as pltpu

P = 8
N = 512
B = 64
EPS = 1e-06


def _body(sp_ref, dst_ref, s_ref, g_ref, out_ref, m_ref):
    i = pl.program_id(0)
    p = sp_ref[i]
    prev = sp_ref[jnp.maximum(i - 1, 0)]
    recompute = jnp.logical_or(i == 0, p != prev)

    @pl.when(recompute)
    def _():
        s = s_ref[0]
        rows = jax.lax.broadcasted_iota(jnp.int32, (N, N), 0)
        cols = jax.lax.broadcasted_iota(jnp.int32, (N, N), 1)
        sz = jnp.where(rows == cols, 0.0, s)
        denom = jnp.clip(jnp.sum(jnp.abs(sz), axis=1, keepdims=True), EPS, None)
        graw = g_ref[0]
        g = jnp.maximum(graw, 0.0) + jnp.log1p(jnp.exp(-jnp.abs(graw))) + 1e-06
        gsum = jnp.clip(jnp.sum(g), EPS, None)
        scale = g * (N / gsum) / denom
        m_ref[...] = sz * scale

    out_ref[0] = m_ref[...]


@jax.jit
def kernel(phases, S, G):
    phases = phases.astype(jnp.int32)
    order = jnp.argsort(phases)
    sorted_phases = phases[order].astype(jnp.int32)
    dest = order.astype(jnp.int32)
    Gc = G.reshape(P, N, 1)

    grid_spec = pltpu.PrefetchScalarGridSpec(
        num_scalar_prefetch=2,
        grid=(B,),
        in_specs=[
            pl.BlockSpec((1, N, N), lambda i, sp, dst: (sp[i], 0, 0)),
            pl.BlockSpec((1, N, 1), lambda i, sp, dst: (sp[i], 0, 0)),
        ],
        out_specs=pl.BlockSpec((1, N, N), lambda i, sp, dst: (dst[i], 0, 0)),
        scratch_shapes=[pltpu.VMEM((N, N), jnp.float32)],
    )

    out = pl.pallas_call(
        _body,
        grid_spec=grid_spec,
        out_shape=jax.ShapeDtypeStruct((B, N, N), jnp.float32),
    )(sorted_phases, dest, S, Gc)
    return out


# speedup vs baseline: 1.5891x; 1.1523x over previous
import jax
import jax.numpy as jnp
from jax.experimental import pallas as pl
from jax.experimental.pallas import tpu as pltpu

P = 8
N = 512
B = 64
EPS = 1e-06


def _body(starts_ref, ends_ref, dst_ref, s_ref, g_ref, out_ref, m_ref, sem):
    i = pl.program_id(0)
    slot = jax.lax.rem(i, 2)

    def wait_phase(p):
        def w(_, c):
            pltpu.make_async_copy(m_ref.at[0], out_ref.at[0], sem).wait()
            return c

        jax.lax.fori_loop(starts_ref[p], ends_ref[p], w, 0)

    @pl.when(i >= 2)
    def _():
        wait_phase(i - 2)

    s = s_ref[0]
    rows = jax.lax.broadcasted_iota(jnp.int32, (N, N), 0)
    cols = jax.lax.broadcasted_iota(jnp.int32, (N, N), 1)
    sz = jnp.where(rows == cols, 0.0, s)
    denom = jnp.clip(jnp.sum(jnp.abs(sz), axis=1, keepdims=True), EPS, None)
    graw = g_ref[0]
    g = jnp.maximum(graw, 0.0) + jnp.log1p(jnp.exp(-jnp.abs(graw))) + 1e-06
    gsum = jnp.clip(jnp.sum(g), EPS, None)
    scale = g * (N / gsum) / denom
    m_ref[slot] = sz * scale

    def issue(k, c):
        pltpu.make_async_copy(m_ref.at[slot], out_ref.at[dst_ref[k]], sem).start()
        return c

    jax.lax.fori_loop(starts_ref[i], ends_ref[i], issue, 0)

    @pl.when(i == P - 1)
    def _():
        wait_phase(P - 2)
        wait_phase(P - 1)


@jax.jit
def kernel(phases, S, G):
    phases = phases.astype(jnp.int32)
    order = jnp.argsort(phases)
    dst = order.astype(jnp.int32)
    counts = jnp.bincount(phases, length=P)
    ends = jnp.cumsum(counts).astype(jnp.int32)
    starts = (ends - counts).astype(jnp.int32)
    Gc = G.reshape(P, N, 1)

    grid_spec = pltpu.PrefetchScalarGridSpec(
        num_scalar_prefetch=3,
        grid=(P,),
        in_specs=[
            pl.BlockSpec((1, N, N), lambda i, st, en, d: (i, 0, 0)),
            pl.BlockSpec((1, N, 1), lambda i, st, en, d: (i, 0, 0)),
        ],
        out_specs=pl.BlockSpec(memory_space=pl.ANY),
        scratch_shapes=[
            pltpu.VMEM((2, N, N), jnp.float32),
            pltpu.SemaphoreType.DMA,
        ],
    )

    out = pl.pallas_call(
        _body,
        grid_spec=grid_spec,
        out_shape=jax.ShapeDtypeStruct((B, N, N), jnp.float32),
    )(starts, ends, dst, S, Gc)
    return out
